# Initial kernel scaffold; baseline (speedup 1.0000x reference)
import jax
import jax.numpy as jnp
from jax.experimental import pallas as pl


def kernel(pos, edge_index, batch, face, Ws, bs):
    raise NotImplementedError("write your pallas kernel here")


# speedup vs baseline: 20.1262x; 20.1262x over previous
import jax
import jax.numpy as jnp
from jax import lax
from jax.experimental import pallas as pl
from jax.experimental.pallas import tpu as pltpu
from jax.experimental.pallas import tpu_sc as plsc

N = 100000
E = 1600000
D = 3
L = 10
W16 = 16

NC = 2
NS = 16
NP = 102400
PAD_NODE = NP - 1
NPF = NP * W16 // 128

EPAD = 1638400
EROWS = EPAD // 128
ROWS_PER_W = EROWS // (NC * NS)
CHUNK_ROWS = 4
N_CHUNKS = ROWS_PER_W // CHUNK_ROWS

NPT = NP // NS
BNC = 400


def _mesh():
    return plsc.VectorSubcoreMesh(
        core_axis_name="c", subcore_axis_name="s", num_cores=NC, num_subcores=NS
    )


_SC_PARAMS = pltpu.CompilerParams(use_tc_tiling_on_sc=False)


def _deg_body(dst2d, ones_hbm, zrow_hbm, deg_out, acc_sp, dstbuf, ones_v,
              bounce, sem):
    cid = lax.axis_index("c")
    sid = lax.axis_index("s")
    wid = cid * NS + sid
    off = sid * NPT

    pltpu.sync_copy(ones_hbm, ones_v)
    pltpu.sync_copy(zrow_hbm, bounce)
    for q in range(NPT // BNC):
        pltpu.sync_copy(bounce, acc_sp.at[pl.ds(off + q * BNC, BNC)])

    plsc.subcore_barrier()

    row0 = wid * ROWS_PER_W

    def chunk(i, _):
        pltpu.sync_copy(dst2d.at[pl.ds(row0 + i * CHUNK_ROWS, CHUNK_ROWS)],
                        dstbuf)
        descs = [
            pltpu.async_copy(ones_v.at[pl.ds(j * 128, 128)],
                             acc_sp.at[dstbuf.at[j]], sem, add=True)
            for j in range(CHUNK_ROWS)
        ]
        for d in descs:
            d.wait()
        return 0
    lax.fori_loop(0, N_CHUNKS, chunk, 0)

    plsc.subcore_barrier()

    for q in range(NPT // BNC):
        pltpu.sync_copy(acc_sp.at[pl.ds(off + q * BNC, BNC)], bounce)
        pltpu.sync_copy(bounce, deg_out.at[cid, pl.ds(off + q * BNC, BNC)])


def _sc_deg(dst2d, ones_hbm, zrow_hbm):
    return pl.kernel(
        _deg_body,
        out_type=[jax.ShapeDtypeStruct((NC, NP, W16), jnp.float32)],
        mesh=_mesh(),
        compiler_params=_SC_PARAMS,
        scratch_types=[
            pltpu.VMEM_SHARED((NP, W16), jnp.float32),
            pltpu.VMEM((CHUNK_ROWS, 128), jnp.int32),
            pltpu.VMEM((CHUNK_ROWS * 128, W16), jnp.float32),
            pltpu.VMEM((BNC, W16), jnp.float32),
            pltpu.SemaphoreType.DMA,
        ],
    )(dst2d, ones_hbm, zrow_hbm)


def _edge_body(g_rows, src2d, dst2d, zrow_hbm, acc_out, acc_sp, srcbuf,
               dstbuf, rows_g, bounce, gsem, ssem):
    cid = lax.axis_index("c")
    sid = lax.axis_index("s")
    wid = cid * NS + sid
    off = sid * NPT

    pltpu.sync_copy(zrow_hbm, bounce)
    for q in range(NPT // BNC):
        pltpu.sync_copy(bounce, acc_sp.at[pl.ds(off + q * BNC, BNC)])

    plsc.subcore_barrier()

    row0 = wid * ROWS_PER_W

    def chunk(i, _):
        pltpu.sync_copy(src2d.at[pl.ds(row0 + i * CHUNK_ROWS, CHUNK_ROWS)],
                        srcbuf)
        pltpu.sync_copy(dst2d.at[pl.ds(row0 + i * CHUNK_ROWS, CHUNK_ROWS)],
                        dstbuf)
        gd = [
            pltpu.async_copy(g_rows.at[srcbuf.at[j]], rows_g.at[j], gsem)
            for j in range(CHUNK_ROWS)
        ]
        for d in gd:
            d.wait()
        sd = [
            pltpu.async_copy(rows_g.at[j], acc_sp.at[dstbuf.at[j]], ssem,
                             add=True)
            for j in range(CHUNK_ROWS)
        ]
        for d in sd:
            d.wait()
        return 0
    lax.fori_loop(0, N_CHUNKS, chunk, 0)

    plsc.subcore_barrier()

    for q in range(NPT // BNC):
        pltpu.sync_copy(acc_sp.at[pl.ds(off + q * BNC, BNC)], bounce)
        pltpu.sync_copy(bounce, acc_out.at[cid, pl.ds(off + q * BNC, BNC)])


def _sc_edge(g_rows, src2d, dst2d, zrow_hbm):
    return pl.kernel(
        _edge_body,
        out_type=[jax.ShapeDtypeStruct((NC, NP, W16), jnp.float32)],
        mesh=_mesh(),
        compiler_params=_SC_PARAMS,
        scratch_types=[
            pltpu.VMEM_SHARED((NP, W16), jnp.float32),
            pltpu.VMEM((CHUNK_ROWS, 128), jnp.int32),
            pltpu.VMEM((CHUNK_ROWS, 128), jnp.int32),
            pltpu.VMEM((CHUNK_ROWS, 128, W16), jnp.float32),
            pltpu.VMEM((BNC, W16), jnp.float32),
            pltpu.SemaphoreType.DMA,
            pltpu.SemaphoreType.DMA,
        ],
    )(g_rows, src2d, dst2d, zrow_hbm)


def _lane_mod(shape):
    return lax.broadcasted_iota(jnp.int32, shape, 1) % W16


def _cvec(m, v0, v1, v2):
    z = jnp.zeros_like(m, dtype=jnp.float32)
    return jnp.where(m == 0, v0, jnp.where(m == 1, v1, jnp.where(m == 2, v2, z)))


def _mix(xn, w_ref):
    m = _lane_mod(xn.shape)
    h = jnp.zeros_like(xn)
    for delta in range(-2, 3):
        vs = []
        for c in range(D):
            k = c - delta
            vs.append(w_ref[k, c] if 0 <= k < D else jnp.float32(0.0))
        coef = _cvec(m, *vs)
        h = h + coef * pltpu.roll(xn, delta % 128, axis=1)
    return h


def _init_body(deg_ref, w_ref, dinv_ref, g_ref):
    deg = deg_ref[0] + deg_ref[1] + 1.0
    dinv = lax.rsqrt(deg)
    dinv_ref[...] = dinv
    m = _lane_mod(deg.shape)
    cs = _cvec(m, w_ref[0, 0] + w_ref[1, 0] + w_ref[2, 0],
               w_ref[0, 1] + w_ref[1, 1] + w_ref[2, 1],
               w_ref[0, 2] + w_ref[1, 2] + w_ref[2, 2])
    p = (lax.broadcasted_iota(jnp.int32, deg.shape, 0) * 128
         + lax.broadcasted_iota(jnp.int32, deg.shape, 1))
    node = p // W16
    ind = jnp.where((node >= 100) & (node < 300), 1.0, 0.0).astype(jnp.float32)
    g_ref[...] = dinv * ind * cs


def _tc_init(deg, w0):
    return pl.pallas_call(
        _init_body,
        out_shape=[
            jax.ShapeDtypeStruct((NPF, 128), jnp.float32),
            jax.ShapeDtypeStruct((NPF, 128), jnp.float32),
        ],
        in_specs=[
            pl.BlockSpec(memory_space=pltpu.VMEM),
            pl.BlockSpec(memory_space=pltpu.SMEM),
        ],
    )(deg, w0)


def _node_body(acc_ref, g_ref, dinv_ref, w_ref, b_ref, out_ref):
    dinv = dinv_ref[...]
    m = _lane_mod(dinv.shape)
    bvec = _cvec(m, b_ref[0, 0], b_ref[0, 1], b_ref[0, 2])
    xn = dinv * (acc_ref[0] + acc_ref[1] + g_ref[...]) + bvec
    out_ref[...] = dinv * _mix(xn, w_ref)


def _tc_node(acc, g, dinv, w, b):
    return pl.pallas_call(
        _node_body,
        out_shape=jax.ShapeDtypeStruct((NPF, 128), jnp.float32),
        in_specs=[
            pl.BlockSpec(memory_space=pltpu.VMEM),
            pl.BlockSpec(memory_space=pltpu.VMEM),
            pl.BlockSpec(memory_space=pltpu.VMEM),
            pl.BlockSpec(memory_space=pltpu.SMEM),
            pl.BlockSpec(memory_space=pltpu.SMEM),
        ],
    )(acc, g, dinv, w, b)


def _final_body(acc_ref, g_ref, dinv_ref, b_ref, out_ref):
    dinv = dinv_ref[...]
    m = _lane_mod(dinv.shape)
    bvec = _cvec(m, b_ref[0, 0], b_ref[0, 1], b_ref[0, 2])
    out_ref[...] = dinv * (acc_ref[0] + acc_ref[1] + g_ref[...]) + bvec


def _tc_final(acc, g, dinv, b):
    return pl.pallas_call(
        _final_body,
        out_shape=jax.ShapeDtypeStruct((NPF, 128), jnp.float32),
        in_specs=[
            pl.BlockSpec(memory_space=pltpu.VMEM),
            pl.BlockSpec(memory_space=pltpu.VMEM),
            pl.BlockSpec(memory_space=pltpu.VMEM),
            pl.BlockSpec(memory_space=pltpu.SMEM),
        ],
    )(acc, g, dinv, b)


def kernel(pos, edge_index, batch, face, Ws, bs):
    src = edge_index[0]
    dst = edge_index[1]
    pad = jnp.full((EPAD - E,), PAD_NODE, jnp.int32)
    src2d = jnp.concatenate([src, pad]).reshape(EROWS, 128)
    dst2d = jnp.concatenate([dst, pad]).reshape(EROWS, 128)

    ones_hbm = jnp.ones((CHUNK_ROWS * 128, W16), jnp.float32)
    zrow_hbm = jnp.zeros((BNC, W16), jnp.float32)

    deg = _sc_deg(dst2d, ones_hbm, zrow_hbm)[0]
    dinv, g = _tc_init(deg.reshape(NC, NPF, 128), Ws[0])

    for l in range(L):
        acc = _sc_edge(g.reshape(NP, W16), src2d, dst2d, zrow_hbm)[0]
        acc = acc.reshape(NC, NPF, 128)
        b = bs[l].reshape(1, D)
        if l < L - 1:
            g = _tc_node(acc, g, dinv, Ws[l + 1], b)
        else:
            xnf = _tc_final(acc, g, dinv, b)

    return xnf.reshape(NP, W16)[:N, :D]


# speedup vs baseline: 23.8100x; 1.1830x over previous
import jax
import jax.numpy as jnp
from jax import lax
from jax.experimental import pallas as pl
from jax.experimental.pallas import tpu as pltpu
from jax.experimental.pallas import tpu_sc as plsc

N = 100000
E = 1600000
D = 3
L = 10
W16 = 16

NC = 2
NS = 16
NP = 102400
PAD_NODE = NP - 1
NPF = NP * W16 // 128

EPAD = 1638400
EROWS = EPAD // 128
ROWS_PER_W = EROWS // (NC * NS)
CHUNK_ROWS = 8
N_CHUNKS = ROWS_PER_W // CHUNK_ROWS

NPT = NP // NS
BNC = 200


def _mesh():
    return plsc.VectorSubcoreMesh(
        core_axis_name="c", subcore_axis_name="s", num_cores=NC, num_subcores=NS
    )


_SC_PARAMS = pltpu.CompilerParams(use_tc_tiling_on_sc=False)


def _deg_body(dst2d, ones_hbm, zrow_hbm, deg_out, acc_sp, dstbuf, ones_v,
              bounce, sem):
    cid = lax.axis_index("c")
    sid = lax.axis_index("s")
    wid = cid * NS + sid
    off = sid * NPT

    pltpu.sync_copy(ones_hbm, ones_v)
    pltpu.sync_copy(zrow_hbm, bounce)
    for q in range(NPT // BNC):
        pltpu.sync_copy(bounce, acc_sp.at[pl.ds(off + q * BNC, BNC)])

    plsc.subcore_barrier()

    row0 = wid * ROWS_PER_W

    def chunk(i, _):
        pltpu.sync_copy(dst2d.at[pl.ds(row0 + i * CHUNK_ROWS, CHUNK_ROWS)],
                        dstbuf)
        descs = [
            pltpu.async_copy(ones_v.at[pl.ds(j * 128, 128)],
                             acc_sp.at[dstbuf.at[j]], sem, add=True)
            for j in range(CHUNK_ROWS)
        ]
        for d in descs:
            d.wait()
        return 0
    lax.fori_loop(0, N_CHUNKS, chunk, 0)

    plsc.subcore_barrier()

    for q in range(NPT // BNC):
        pltpu.sync_copy(acc_sp.at[pl.ds(off + q * BNC, BNC)], bounce)
        pltpu.sync_copy(bounce, deg_out.at[cid, pl.ds(off + q * BNC, BNC)])


def _sc_deg(dst2d, ones_hbm, zrow_hbm):
    return pl.kernel(
        _deg_body,
        out_type=[jax.ShapeDtypeStruct((NC, NP, W16), jnp.float32)],
        mesh=_mesh(),
        compiler_params=_SC_PARAMS,
        scratch_types=[
            pltpu.VMEM_SHARED((NP, W16), jnp.float32),
            pltpu.VMEM((CHUNK_ROWS, 128), jnp.int32),
            pltpu.VMEM((CHUNK_ROWS * 128, W16), jnp.float32),
            pltpu.VMEM((BNC, W16), jnp.float32),
            pltpu.SemaphoreType.DMA,
        ],
    )(dst2d, ones_hbm, zrow_hbm)


def _edge_body(g_rows, comb, zrow_hbm, acc_out, acc_sp, idxbuf,
               rows_g, bounce, gsem, ssem):
    cid = lax.axis_index("c")
    sid = lax.axis_index("s")
    wid = cid * NS + sid
    off = sid * NPT

    pltpu.sync_copy(zrow_hbm, bounce)
    for q in range(NPT // BNC):
        pltpu.sync_copy(bounce, acc_sp.at[pl.ds(off + q * BNC, BNC)])

    plsc.subcore_barrier()


    def chunk(i, _):
        pltpu.sync_copy(comb.at[wid, i], idxbuf)
        gd = [
            pltpu.async_copy(g_rows.at[idxbuf.at[j]], rows_g.at[j], gsem)
            for j in range(CHUNK_ROWS)
        ]
        for d in gd:
            d.wait()
        sd = [
            pltpu.async_copy(rows_g.at[j], acc_sp.at[idxbuf.at[CHUNK_ROWS + j]],
                             ssem, add=True)
            for j in range(CHUNK_ROWS)
        ]
        for d in sd:
            d.wait()
        return 0
    lax.fori_loop(0, N_CHUNKS, chunk, 0)

    plsc.subcore_barrier()

    for q in range(NPT // BNC):
        pltpu.sync_copy(acc_sp.at[pl.ds(off + q * BNC, BNC)], bounce)
        pltpu.sync_copy(bounce, acc_out.at[cid, pl.ds(off + q * BNC, BNC)])


def _sc_edge(g_rows, comb, zrow_hbm):
    return pl.kernel(
        _edge_body,
        out_type=[jax.ShapeDtypeStruct((NC, NP, W16), jnp.float32)],
        mesh=_mesh(),
        compiler_params=_SC_PARAMS,
        scratch_types=[
            pltpu.VMEM_SHARED((NP, W16), jnp.float32),
            pltpu.VMEM((2 * CHUNK_ROWS, 128), jnp.int32),
            pltpu.VMEM((CHUNK_ROWS, 128, W16), jnp.float32),
            pltpu.VMEM((BNC, W16), jnp.float32),
            pltpu.SemaphoreType.DMA,
            pltpu.SemaphoreType.DMA,
        ],
    )(g_rows, comb, zrow_hbm)


def _lane_mod(shape):
    return lax.broadcasted_iota(jnp.int32, shape, 1) % W16


def _cvec(m, v0, v1, v2):
    z = jnp.zeros_like(m, dtype=jnp.float32)
    return jnp.where(m == 0, v0, jnp.where(m == 1, v1, jnp.where(m == 2, v2, z)))


def _mix(xn, w_ref):
    m = _lane_mod(xn.shape)
    h = jnp.zeros_like(xn)
    for delta in range(-2, 3):
        vs = []
        for c in range(D):
            k = c - delta
            vs.append(w_ref[k, c] if 0 <= k < D else jnp.float32(0.0))
        coef = _cvec(m, *vs)
        h = h + coef * pltpu.roll(xn, delta % 128, axis=1)
    return h


def _init_body(deg_ref, w_ref, dinv_ref, g_ref):
    deg = deg_ref[0] + deg_ref[1] + 1.0
    dinv = lax.rsqrt(deg)
    dinv_ref[...] = dinv
    m = _lane_mod(deg.shape)
    cs = _cvec(m, w_ref[0, 0] + w_ref[1, 0] + w_ref[2, 0],
               w_ref[0, 1] + w_ref[1, 1] + w_ref[2, 1],
               w_ref[0, 2] + w_ref[1, 2] + w_ref[2, 2])
    p = (lax.broadcasted_iota(jnp.int32, deg.shape, 0) * 128
         + lax.broadcasted_iota(jnp.int32, deg.shape, 1))
    node = p // W16
    ind = jnp.where((node >= 100) & (node < 300), 1.0, 0.0).astype(jnp.float32)
    g_ref[...] = dinv * ind * cs


def _tc_init(deg, w0):
    return pl.pallas_call(
        _init_body,
        out_shape=[
            jax.ShapeDtypeStruct((NPF, 128), jnp.float32),
            jax.ShapeDtypeStruct((NPF, 128), jnp.float32),
        ],
        in_specs=[
            pl.BlockSpec(memory_space=pltpu.VMEM),
            pl.BlockSpec(memory_space=pltpu.SMEM),
        ],
    )(deg, w0)


def _node_body(acc_ref, g_ref, dinv_ref, w_ref, b_ref, out_ref):
    dinv = dinv_ref[...]
    m = _lane_mod(dinv.shape)
    bvec = _cvec(m, b_ref[0, 0], b_ref[0, 1], b_ref[0, 2])
    xn = dinv * (acc_ref[0] + acc_ref[1] + g_ref[...]) + bvec
    out_ref[...] = dinv * _mix(xn, w_ref)


def _tc_node(acc, g, dinv, w, b):
    return pl.pallas_call(
        _node_body,
        out_shape=jax.ShapeDtypeStruct((NPF, 128), jnp.float32),
        in_specs=[
            pl.BlockSpec(memory_space=pltpu.VMEM),
            pl.BlockSpec(memory_space=pltpu.VMEM),
            pl.BlockSpec(memory_space=pltpu.VMEM),
            pl.BlockSpec(memory_space=pltpu.SMEM),
            pl.BlockSpec(memory_space=pltpu.SMEM),
        ],
    )(acc, g, dinv, w, b)


def _final_body(acc_ref, g_ref, dinv_ref, b_ref, out_ref):
    dinv = dinv_ref[...]
    m = _lane_mod(dinv.shape)
    bvec = _cvec(m, b_ref[0, 0], b_ref[0, 1], b_ref[0, 2])
    out_ref[...] = dinv * (acc_ref[0] + acc_ref[1] + g_ref[...]) + bvec


def _tc_final(acc, g, dinv, b):
    return pl.pallas_call(
        _final_body,
        out_shape=jax.ShapeDtypeStruct((NPF, 128), jnp.float32),
        in_specs=[
            pl.BlockSpec(memory_space=pltpu.VMEM),
            pl.BlockSpec(memory_space=pltpu.VMEM),
            pl.BlockSpec(memory_space=pltpu.VMEM),
            pl.BlockSpec(memory_space=pltpu.SMEM),
        ],
    )(acc, g, dinv, b)


def kernel(pos, edge_index, batch, face, Ws, bs):
    src = edge_index[0]
    dst = edge_index[1]
    pad = jnp.full((EPAD - E,), PAD_NODE, jnp.int32)
    src2d = jnp.concatenate([src, pad]).reshape(EROWS, 128)
    dst2d = jnp.concatenate([dst, pad]).reshape(EROWS, 128)

    ones_hbm = jnp.ones((CHUNK_ROWS * 128, W16), jnp.float32)
    zrow_hbm = jnp.zeros((BNC, W16), jnp.float32)

    src3 = src2d.reshape(NC * NS, N_CHUNKS, CHUNK_ROWS, 128)
    dst3 = dst2d.reshape(NC * NS, N_CHUNKS, CHUNK_ROWS, 128)
    comb = jnp.concatenate([src3, dst3], axis=2)

    deg = _sc_deg(dst2d, ones_hbm, zrow_hbm)[0]
    dinv, g = _tc_init(deg.reshape(NC, NPF, 128), Ws[0])

    for l in range(L):
        acc = _sc_edge(g.reshape(NP, W16), comb, zrow_hbm)[0]
        acc = acc.reshape(NC, NPF, 128)
        b = bs[l].reshape(1, D)
        if l < L - 1:
            g = _tc_node(acc, g, dinv, Ws[l + 1], b)
        else:
            xnf = _tc_final(acc, g, dinv, b)

    return xnf.reshape(NP, W16)[:N, :D]


# speedup vs baseline: 25.2153x; 1.0590x over previous
import jax
import jax.numpy as jnp
from jax import lax
from jax.experimental import pallas as pl
from jax.experimental.pallas import tpu as pltpu
from jax.experimental.pallas import tpu_sc as plsc

N = 100000
E = 1600000
D = 3
L = 10
W16 = 16

NC = 2
NS = 16
NP = 102400
PAD_NODE = NP - 1
NPF = NP * W16 // 128

EPAD = 1638400
EROWS = EPAD // 128
ROWS_PER_W = EROWS // (NC * NS)
CHUNK_ROWS = 4
N_CHUNKS = ROWS_PER_W // CHUNK_ROWS
NSLOT = 2

NPT = NP // NS
BNC = 200


def _mesh():
    return plsc.VectorSubcoreMesh(
        core_axis_name="c", subcore_axis_name="s", num_cores=NC, num_subcores=NS
    )


_SC_PARAMS = pltpu.CompilerParams(use_tc_tiling_on_sc=False)


def _deg_body(dst2d, ones_hbm, zrow_hbm, deg_out, acc_sp, dstbuf, ones_v,
              bounce, sem):
    cid = lax.axis_index("c")
    sid = lax.axis_index("s")
    wid = cid * NS + sid
    off = sid * NPT

    pltpu.sync_copy(ones_hbm, ones_v)
    pltpu.sync_copy(zrow_hbm, bounce)
    for q in range(NPT // BNC):
        pltpu.sync_copy(bounce, acc_sp.at[pl.ds(off + q * BNC, BNC)])

    plsc.subcore_barrier()

    row0 = wid * ROWS_PER_W

    def chunk(i, _):
        pltpu.sync_copy(dst2d.at[pl.ds(row0 + i * CHUNK_ROWS, CHUNK_ROWS)],
                        dstbuf)
        descs = [
            pltpu.async_copy(ones_v.at[pl.ds(j * 128, 128)],
                             acc_sp.at[dstbuf.at[j]], sem, add=True)
            for j in range(CHUNK_ROWS)
        ]
        for d in descs:
            d.wait()
        return 0
    lax.fori_loop(0, N_CHUNKS, chunk, 0)

    plsc.subcore_barrier()

    for q in range(NPT // BNC):
        pltpu.sync_copy(acc_sp.at[pl.ds(off + q * BNC, BNC)], bounce)
        pltpu.sync_copy(bounce, deg_out.at[cid, pl.ds(off + q * BNC, BNC)])


def _sc_deg(dst2d, ones_hbm, zrow_hbm):
    return pl.kernel(
        _deg_body,
        out_type=[jax.ShapeDtypeStruct((NC, NP, W16), jnp.float32)],
        mesh=_mesh(),
        compiler_params=_SC_PARAMS,
        scratch_types=[
            pltpu.VMEM_SHARED((NP, W16), jnp.float32),
            pltpu.VMEM((CHUNK_ROWS, 128), jnp.int32),
            pltpu.VMEM((CHUNK_ROWS * 128, W16), jnp.float32),
            pltpu.VMEM((BNC, W16), jnp.float32),
            pltpu.SemaphoreType.DMA,
        ],
    )(dst2d, ones_hbm, zrow_hbm)


def _edge_body(g_rows, comb, zrow_hbm, acc_out, acc_sp, idx0, idx1, rows0,
               rows1, bounce, gsem0, gsem1, ssem0, ssem1):
    cid = lax.axis_index("c")
    sid = lax.axis_index("s")
    wid = cid * NS + sid
    off = sid * NPT
    idxs = [idx0, idx1]
    rows = [rows0, rows1]
    gsems = [gsem0, gsem1]
    ssems = [ssem0, ssem1]

    pltpu.sync_copy(zrow_hbm, bounce)
    for q in range(NPT // BNC):
        pltpu.sync_copy(bounce, acc_sp.at[pl.ds(off + q * BNC, BNC)])

    plsc.subcore_barrier()


    def fire_gathers(c, r):
        pltpu.sync_copy(comb.at[wid, c], idxs[r])
        return [
            pltpu.async_copy(g_rows.at[idxs[r].at[j]], rows[r].at[j], gsems[r])
            for j in range(CHUNK_ROWS)
        ]

    def fire_scatters(r):
        return [
            pltpu.async_copy(rows[r].at[j],
                             acc_sp.at[idxs[r].at[CHUNK_ROWS + j]],
                             ssems[r], add=True)
            for j in range(CHUNK_ROWS)
        ]

    def pair(t, _):
        gd0 = fire_gathers(2 * t, 0)
        gd1 = fire_gathers(2 * t + 1, 1)
        for d in gd0:
            d.wait()
        sd0 = fire_scatters(0)
        for d in gd1:
            d.wait()
        sd1 = fire_scatters(1)
        for d in sd0:
            d.wait()
        for d in sd1:
            d.wait()
        return 0
    lax.fori_loop(0, N_CHUNKS // 2, pair, 0)

    plsc.subcore_barrier()

    for q in range(NPT // BNC):
        pltpu.sync_copy(acc_sp.at[pl.ds(off + q * BNC, BNC)], bounce)
        pltpu.sync_copy(bounce, acc_out.at[cid, pl.ds(off + q * BNC, BNC)])


def _sc_edge(g_rows, comb, zrow_hbm):
    return pl.kernel(
        _edge_body,
        out_type=[jax.ShapeDtypeStruct((NC, NP, W16), jnp.float32)],
        mesh=_mesh(),
        compiler_params=_SC_PARAMS,
        scratch_types=[
            pltpu.VMEM_SHARED((NP, W16), jnp.float32),
            pltpu.VMEM((2 * CHUNK_ROWS, 128), jnp.int32),
            pltpu.VMEM((2 * CHUNK_ROWS, 128), jnp.int32),
            pltpu.VMEM((CHUNK_ROWS, 128, W16), jnp.float32),
            pltpu.VMEM((CHUNK_ROWS, 128, W16), jnp.float32),
            pltpu.VMEM((BNC, W16), jnp.float32),
            pltpu.SemaphoreType.DMA,
            pltpu.SemaphoreType.DMA,
            pltpu.SemaphoreType.DMA,
            pltpu.SemaphoreType.DMA,
        ],
    )(g_rows, comb, zrow_hbm)


def _lane_mod(shape):
    return lax.broadcasted_iota(jnp.int32, shape, 1) % W16


def _cvec(m, v0, v1, v2):
    z = jnp.zeros_like(m, dtype=jnp.float32)
    return jnp.where(m == 0, v0, jnp.where(m == 1, v1, jnp.where(m == 2, v2, z)))


def _mix(xn, w_ref):
    m = _lane_mod(xn.shape)
    h = jnp.zeros_like(xn)
    for delta in range(-2, 3):
        vs = []
        for c in range(D):
            k = c - delta
            vs.append(w_ref[k, c] if 0 <= k < D else jnp.float32(0.0))
        coef = _cvec(m, *vs)
        h = h + coef * pltpu.roll(xn, delta % 128, axis=1)
    return h


def _init_body(deg_ref, w_ref, dinv_ref, g_ref):
    deg = deg_ref[0] + deg_ref[1] + 1.0
    dinv = lax.rsqrt(deg)
    dinv_ref[...] = dinv
    m = _lane_mod(deg.shape)
    cs = _cvec(m, w_ref[0, 0] + w_ref[1, 0] + w_ref[2, 0],
               w_ref[0, 1] + w_ref[1, 1] + w_ref[2, 1],
               w_ref[0, 2] + w_ref[1, 2] + w_ref[2, 2])
    p = (lax.broadcasted_iota(jnp.int32, deg.shape, 0) * 128
         + lax.broadcasted_iota(jnp.int32, deg.shape, 1))
    node = p // W16
    ind = jnp.where((node >= 100) & (node < 300), 1.0, 0.0).astype(jnp.float32)
    g_ref[...] = dinv * ind * cs


def _tc_init(deg, w0):
    return pl.pallas_call(
        _init_body,
        out_shape=[
            jax.ShapeDtypeStruct((NPF, 128), jnp.float32),
            jax.ShapeDtypeStruct((NPF, 128), jnp.float32),
        ],
        in_specs=[
            pl.BlockSpec(memory_space=pltpu.VMEM),
            pl.BlockSpec(memory_space=pltpu.SMEM),
        ],
    )(deg, w0)


def _node_body(acc_ref, g_ref, dinv_ref, w_ref, b_ref, out_ref):
    dinv = dinv_ref[...]
    m = _lane_mod(dinv.shape)
    bvec = _cvec(m, b_ref[0, 0], b_ref[0, 1], b_ref[0, 2])
    xn = dinv * (acc_ref[0] + acc_ref[1] + g_ref[...]) + bvec
    out_ref[...] = dinv * _mix(xn, w_ref)


def _tc_node(acc, g, dinv, w, b):
    return pl.pallas_call(
        _node_body,
        out_shape=jax.ShapeDtypeStruct((NPF, 128), jnp.float32),
        in_specs=[
            pl.BlockSpec(memory_space=pltpu.VMEM),
            pl.BlockSpec(memory_space=pltpu.VMEM),
            pl.BlockSpec(memory_space=pltpu.VMEM),
            pl.BlockSpec(memory_space=pltpu.SMEM),
            pl.BlockSpec(memory_space=pltpu.SMEM),
        ],
    )(acc, g, dinv, w, b)


def _final_body(acc_ref, g_ref, dinv_ref, b_ref, out_ref):
    dinv = dinv_ref[...]
    m = _lane_mod(dinv.shape)
    bvec = _cvec(m, b_ref[0, 0], b_ref[0, 1], b_ref[0, 2])
    out_ref[...] = dinv * (acc_ref[0] + acc_ref[1] + g_ref[...]) + bvec


def _tc_final(acc, g, dinv, b):
    return pl.pallas_call(
        _final_body,
        out_shape=jax.ShapeDtypeStruct((NPF, 128), jnp.float32),
        in_specs=[
            pl.BlockSpec(memory_space=pltpu.VMEM),
            pl.BlockSpec(memory_space=pltpu.VMEM),
            pl.BlockSpec(memory_space=pltpu.VMEM),
            pl.BlockSpec(memory_space=pltpu.SMEM),
        ],
    )(acc, g, dinv, b)


def kernel(pos, edge_index, batch, face, Ws, bs):
    src = edge_index[0]
    dst = edge_index[1]
    pad = jnp.full((EPAD - E,), PAD_NODE, jnp.int32)
    src2d = jnp.concatenate([src, pad]).reshape(EROWS, 128)
    dst2d = jnp.concatenate([dst, pad]).reshape(EROWS, 128)

    ones_hbm = jnp.ones((CHUNK_ROWS * 128, W16), jnp.float32)
    zrow_hbm = jnp.zeros((BNC, W16), jnp.float32)

    src3 = src2d.reshape(NC * NS, N_CHUNKS, CHUNK_ROWS, 128)
    dst3 = dst2d.reshape(NC * NS, N_CHUNKS, CHUNK_ROWS, 128)
    comb = jnp.concatenate([src3, dst3], axis=2)

    deg = _sc_deg(dst2d, ones_hbm, zrow_hbm)[0]
    dinv, g = _tc_init(deg.reshape(NC, NPF, 128), Ws[0])

    for l in range(L):
        acc = _sc_edge(g.reshape(NP, W16), comb, zrow_hbm)[0]
        acc = acc.reshape(NC, NPF, 128)
        b = bs[l].reshape(1, D)
        if l < L - 1:
            g = _tc_node(acc, g, dinv, Ws[l + 1], b)
        else:
            xnf = _tc_final(acc, g, dinv, b)

    return xnf.reshape(NP, W16)[:N, :D]
